# baseline (speedup 1.0000x reference)
import functools

import jax
import jax.numpy as jnp
import numpy as np
from jax import lax
from jax.experimental import pallas as pl
from jax.experimental.pallas import tpu as pltpu
from jax.experimental.pallas import tpu_sc as plsc

_NC = 2
_NS = 16
_NW = _NC * _NS

B = 32
N = 32
NTOT = B * N
CUTOFF = 6.0
NBLK = 3
RBF = 16
HID = 128
DNODE = 64
DMSG = 16

_DELTA = CUTOFF / (RBF - 1)
_GAMMA = 1.0 / (_DELTA * _DELTA + 1e-12)

_R_NP = np.zeros((DNODE, DNODE * DMSG), np.float32)
for _i in range(DNODE):
    _R_NP[_i, _i * DMSG:(_i + 1) * DMSG] = 1.0
_S_NP = np.zeros((DNODE * DMSG, DMSG), np.float32)
for _j in range(DNODE * DMSG):
    _S_NP[_j, _j % DMSG] = 1.0

_TE = 1024

_COSPOLY = (1.00000000e+00, -4.93480220e+00, 4.05871213e+00,
            -1.33526277e+00, 2.35330630e-01, -2.58068912e-02,
            1.92957371e-03, -1.04637021e-04, 4.30179273e-06,
            -1.37843422e-07, 3.20231643e-09)


def _silu(x):
    return x / (1.0 + jnp.exp(-x))


def _full(shape):
    return pl.BlockSpec(shape, lambda i: (0, 0))


def _geom_body(E, es_ref, ed_ref, poshi_ref, poslo_ref, nid_ref, out_ref):
    i = pl.program_id(0)
    bf16 = jnp.bfloat16
    f32 = jnp.float32
    nid = nid_ref[...]
    dsel = ((nid == ed_ref[...]).astype(bf16)
            - (nid == es_ref[...]).astype(bf16))
    d = (jnp.dot(dsel, poshi_ref[...], preferred_element_type=f32)
         + jnp.dot(dsel, poslo_ref[...], preferred_element_type=f32))
    r = jnp.sqrt(jnp.sum(d * d, axis=1, keepdims=True) + 1e-16)
    centers = lax.broadcasted_iota(
        jnp.int32, (_TE, RBF), 1).astype(jnp.float32) * _DELTA
    rbf = jnp.exp(-_GAMMA * (r - centers) ** 2)
    row = lax.broadcasted_iota(jnp.int32, (_TE, 1), 0) + i * _TE
    valid = (row < E).astype(jnp.float32)
    u = r * r * (1.0 / (CUTOFF * CUTOFF))
    c = jnp.float32(_COSPOLY[-1])
    for a in _COSPOLY[-2::-1]:
        c = c * u + jnp.float32(a)
    cut = 0.5 * (c + 1.0)
    cut = cut * (r <= CUTOFF).astype(jnp.float32) * valid
    out_ref[...] = jnp.concatenate(
        [rbf, jnp.broadcast_to(cut, (_TE, RBF))], axis=1).astype(jnp.bfloat16)


def _geom(esc, edc, poshi, poslo, nid, E, Epad):
    return pl.pallas_call(
        functools.partial(_geom_body, E),
        grid=(Epad // _TE,),
        in_specs=[pl.BlockSpec((_TE, 1), lambda i: (i, 0)),
                  pl.BlockSpec((_TE, 1), lambda i: (i, 0)),
                  pl.BlockSpec((NTOT, 16), lambda i: (0, 0)),
                  pl.BlockSpec((NTOT, 16), lambda i: (0, 0)),
                  pl.BlockSpec((1, NTOT), lambda i: (0, 0))],
        out_specs=pl.BlockSpec((_TE, 2 * RBF), lambda i: (i, 0)),
        out_shape=jax.ShapeDtypeStruct((Epad, 2 * RBF), jnp.bfloat16),
    )(esc, edc, poshi, poslo, nid)


def _edge_body(rc_ref, es_ref, xn_ref, nid_ref, rw1, rw2, rw3,
               gw1, gw2, rm, sm, out_ref):
    f32 = jnp.float32
    bf16 = jnp.bfloat16
    rc = rc_ref[...]
    rbf = rc[:, :RBF]
    cut = rc[:, RBF:RBF + 1].astype(f32)
    h = _silu(jnp.dot(rbf, rw1[...], preferred_element_type=f32))
    h = _silu(jnp.dot(h.astype(bf16), rw2[...], preferred_element_type=f32))
    tw = jnp.dot(h.astype(bf16), rw3[...], preferred_element_type=f32)
    gs = (nid_ref[...] == es_ref[...]).astype(bf16)
    xse = jnp.dot(gs, xn_ref[...][:, :DNODE].astype(bf16),
                  preferred_element_type=f32)
    xr = jnp.dot(xse.astype(bf16), rm[...], preferred_element_type=f32)
    m = jnp.dot((tw * xr).astype(bf16), sm[...],
                preferred_element_type=f32) * 0.125
    g = _silu(jnp.dot(rbf, gw1[...], preferred_element_type=f32))
    gz = jnp.dot(g.astype(bf16), gw2[...], preferred_element_type=f32)
    sg = 1.0 / (1.0 + jnp.exp(-gz[:, :1]))
    ew = cut * sg
    out_ref[...] = jnp.concatenate(
        [m * ew, jnp.broadcast_to(ew, (_TE, 16)),
         jnp.zeros((_TE, 96), f32)], axis=1)


def _edge(rc, esc, xn128, nid, w, e0, Ec):
    (rw1, rw2, rw3, gw1, gw2, rm, sm) = w
    b0 = e0 // _TE
    return pl.pallas_call(
        _edge_body,
        grid=(Ec // _TE,),
        in_specs=[pl.BlockSpec((_TE, 2 * RBF), lambda i: (b0 + i, 0)),
                  pl.BlockSpec((_TE, 1), lambda i: (b0 + i, 0)),
                  _full((NTOT, 128)),
                  pl.BlockSpec((1, NTOT), lambda i: (0, 0)),
                  _full((RBF, HID)),
                  _full((HID, HID)),
                  _full((HID, DNODE * DMSG)),
                  _full((RBF, HID)),
                  _full((HID, HID)),
                  _full((DNODE, DNODE * DMSG)),
                  _full((DNODE * DMSG, DMSG))],
        out_specs=pl.BlockSpec((_TE, 128), lambda i: (i, 0)),
        out_shape=jax.ShapeDtypeStruct((Ec, 128), jnp.float32),
    )(rc, esc, xn128, nid, rw1, rw2, rw3, gw1, gw2, rm, sm)


def _ln(x, w, b):
    mu = jnp.mean(x, axis=1, keepdims=True)
    var = jnp.mean((x - mu) ** 2, axis=1, keepdims=True)
    return (x - mu) / jnp.sqrt(var + 1e-8) * w + b


def _ln0_body(x_ref, w_ref, b_ref, out_ref):
    xn = _ln(x_ref[...], w_ref[...], b_ref[...])
    out_ref[...] = jnp.concatenate(
        [xn, jnp.zeros((NTOT, 128 - DNODE), jnp.float32)], axis=1)


def _ln0(x, w, b):
    return pl.pallas_call(
        _ln0_body,
        out_shape=jax.ShapeDtypeStruct((NTOT, 128), jnp.float32),
    )(x, w, b)


def _upd_core(agg2a_ref, agg2b_ref, x_ref, xn_ref, wmsg, wupd, wself,
              rs_ref):
    f32 = jnp.float32
    s = (agg2a_ref[0] + agg2a_ref[1]) + (agg2b_ref[0] + agg2b_ref[1])
    agg = s[:, :DMSG]
    nrm = s[:, DMSG:DMSG + 1]
    agg = agg / jnp.maximum(nrm, 1e-8)
    agg = _silu(jnp.dot(agg, wmsg[...], preferred_element_type=f32) * 0.25)
    xn = xn_ref[...][:, :DNODE]
    out = (jnp.dot(xn, wself[...], preferred_element_type=f32) * 0.125
           + jnp.dot(agg, wupd[...], preferred_element_type=f32) * 0.25)
    return x_ref[...] + rs_ref[0, 0] * out


def _upd_mid_body(agg2a_ref, agg2b_ref, x_ref, xn_ref, wmsg, wupd, wself,
                  rs_ref, pnw_ref, pnb_ref, xo_ref, xno_ref):
    xnew = _upd_core(agg2a_ref, agg2b_ref, x_ref, xn_ref, wmsg, wupd,
                     wself, rs_ref)
    xo_ref[...] = xnew
    xn2 = _ln(xnew, pnw_ref[...], pnb_ref[...])
    xno_ref[...] = jnp.concatenate(
        [xn2, jnp.zeros((NTOT, 128 - DNODE), jnp.float32)], axis=1)


def _upd_last_body(agg2a_ref, agg2b_ref, x_ref, xn_ref, wmsg, wupd, wself,
                   rs_ref, mask_ref, y_ref):
    xnew = _upd_core(agg2a_ref, agg2b_ref, x_ref, xn_ref, wmsg, wupd,
                     wself, rs_ref)
    y_ref[...] = xnew * mask_ref[...]


def _upd_mid(agg2a, agg2b, x, xn128, wmsg, wupd, wself, rs, pnw, pnb):
    return pl.pallas_call(
        _upd_mid_body,
        out_shape=[jax.ShapeDtypeStruct((NTOT, DNODE), jnp.float32),
                   jax.ShapeDtypeStruct((NTOT, 128), jnp.float32)],
    )(agg2a, agg2b, x, xn128, wmsg, wupd, wself, rs, pnw, pnb)


def _upd_last(agg2a, agg2b, x, xn128, wmsg, wupd, wself, rs, maskb):
    return pl.pallas_call(
        _upd_last_body,
        out_shape=jax.ShapeDtypeStruct((NTOT, DNODE), jnp.float32),
    )(agg2a, agg2b, x, xn128, wmsg, wupd, wself, rs, maskb)


def _sc_mesh():
    return plsc.VectorSubcoreMesh(core_axis_name="c", subcore_axis_name="s",
                                  num_cores=_NC, num_subcores=_NS)


def _wid():
    return lax.axis_index("s") * _NC + lax.axis_index("c")


def _sc_prep(embed128, zf):
    f32 = jnp.float32
    nz = NTOT // _NW

    @functools.partial(
        pl.kernel,
        out_type=jax.ShapeDtypeStruct((NTOT, 128), f32),
        mesh=_sc_mesh(),
        scratch_types=[pltpu.VMEM((nz,), jnp.int32),
                       pltpu.VMEM((nz, 128), f32),
                       pltpu.SemaphoreType.DMA],
    )
    def prep(embed_h, zf_h, x0_h, zbuf, zrows, sem):
        wid = _wid()
        pltpu.sync_copy(zf_h.at[pl.ds(wid * nz, nz)], zbuf)
        pltpu.async_copy(embed_h.at[zbuf], zrows, sem).wait()
        pltpu.sync_copy(zrows, x0_h.at[pl.ds(wid * nz, nz)])

    return prep(embed128, zf)


def _sc_scatter(medge, edp2, zeros128, e0, Ec):
    K = Ec // (_NW * 128)
    r0 = e0 // 128
    f32 = jnp.float32
    nrows = NTOT // _NS

    @functools.partial(
        pl.kernel,
        out_type=jax.ShapeDtypeStruct((_NC, NTOT, 128), f32),
        mesh=_sc_mesh(),
        scratch_types=[pltpu.VMEM_SHARED((NTOT, 128), f32),
                       pltpu.VMEM((K, 128), jnp.int32),
                       pltpu.VMEM((128, 128), f32),
                       pltpu.VMEM((128, 128), f32),
                       pltpu.SemaphoreType.DMA,
                       pltpu.SemaphoreType.DMA],
    )
    def scat(m_h, ed_h, z_h, out_h, shared, idx_v, mb0, mb1, lsem, asem):
        cid = lax.axis_index("c")
        sid = lax.axis_index("s")
        wid = sid * _NC + cid
        bufs = (mb0, mb1)
        pltpu.sync_copy(z_h.at[pl.ds(sid * nrows, nrows)],
                        shared.at[pl.ds(sid * nrows, nrows)])
        plsc.subcore_barrier()
        pltpu.sync_copy(ed_h.at[pl.ds(r0 + wid * K, K)], idx_v)
        ld = pltpu.async_copy(m_h.at[pl.ds(wid * K * 128, 128)], bufs[0], lsem)
        ad = None
        for j in range(K):
            ld.wait()
            if ad is not None:
                ad.wait()
            if j + 1 < K:
                ld = pltpu.async_copy(
                    m_h.at[pl.ds((wid * K + j + 1) * 128, 128)],
                    bufs[(j + 1) % 2], lsem)
            ad = pltpu.async_copy(
                bufs[j % 2], shared.at[idx_v.at[j]], asem, add=True)
        ad.wait()
        plsc.subcore_barrier()
        pltpu.sync_copy(shared.at[pl.ds(sid * nrows, nrows)],
                        out_h.at[cid, pl.ds(sid * nrows, nrows)])

    return scat(medge, edp2, zeros128)


def kernel(z, pos, mask, edge_src, edge_dst, embed,
           pnw0, pnb0, rw10, rb10, rw20, rb20, rw30, rb30,
           gw10, gb10, gw20, gb20, wmsg0, wupd0, wself0, rs0,
           pnw1, pnb1, rw11, rb11, rw21, rb21, rw31, rb31,
           gw11, gb11, gw21, gb21, wmsg1, wupd1, wself1, rs1,
           pnw2, pnb2, rw12, rb12, rw22, rb22, rw32, rb32,
           gw12, gb12, gw22, gb22, wmsg2, wupd2, wself2, rs2):
    f32 = jnp.float32
    E = edge_src.shape[0]
    Epad = max(((E + 8191) // 8192) * 8192, 8192)
    Ec = Epad // 2
    esp = jnp.pad(edge_src.astype(jnp.int32), (0, Epad - E))
    edp = jnp.pad(edge_dst.astype(jnp.int32), (0, Epad - E))
    edp2 = edp.reshape(-1, 128)
    esc = esp.reshape(-1, 1)
    edc = edp.reshape(-1, 1)
    posf = pos.reshape(NTOT, 3).astype(f32)
    posp = jnp.pad(posf, ((0, 0), (0, 13)))
    poshi = posp.astype(jnp.bfloat16)
    poslo = (posp - poshi.astype(f32)).astype(jnp.bfloat16)
    embed128 = jnp.pad(embed.astype(f32), ((0, 0), (0, 128 - DNODE)))
    zf = z.reshape(-1).astype(jnp.int32)
    zeros128 = jnp.zeros((NTOT, 128), f32)
    maskb = jnp.broadcast_to(mask.reshape(NTOT, 1), (NTOT, DNODE)).astype(f32)
    rm = jnp.asarray(_R_NP, dtype=jnp.bfloat16)
    sm = jnp.asarray(_S_NP, dtype=jnp.bfloat16)
    nid = jnp.arange(NTOT, dtype=jnp.int32).reshape(1, NTOT)

    blocks = [
        (pnw0, pnb0, rw10, rb10, rw20, rb20, rw30, rb30, gw10, gb10, gw20,
         gb20, wmsg0, wupd0, wself0, rs0),
        (pnw1, pnb1, rw11, rb11, rw21, rb21, rw31, rb31, gw11, gb11, gw21,
         gb21, wmsg1, wupd1, wself1, rs1),
        (pnw2, pnb2, rw12, rb12, rw22, rb22, rw32, rb32, gw12, gb12, gw22,
         gb22, wmsg2, wupd2, wself2, rs2),
    ]

    x0128 = _sc_prep(embed128, zf)
    x = x0128[:, :DNODE]

    rc = _geom(esc, edc, poshi, poslo, nid, E, Epad)

    xn128 = _ln0(x, pnw0.reshape(1, DNODE), pnb0.reshape(1, DNODE))
    for i in range(NBLK):
        (pnw, pnb, rw1, rb1, rw2, rb2, rw3, rb3, gw1, gb1, gw2, gb2,
         wmsg, wupd, wself, rs) = blocks[i]
        bf16 = jnp.bfloat16
        gw2p = jnp.pad(gw2, ((0, 0), (0, HID - 1))).astype(bf16)
        w = (rw1.astype(bf16), rw2.astype(bf16), rw3.astype(bf16),
             gw1.astype(bf16), gw2p, rm, sm)
        medge0 = _edge(rc, esc, xn128, nid, w, 0, Ec)
        agg2a = _sc_scatter(medge0, edp2, zeros128, 0, Ec)
        medge1 = _edge(rc, esc, xn128, nid, w, Ec, Ec)
        agg2b = _sc_scatter(medge1, edp2, zeros128, Ec, Ec)
        if i < NBLK - 1:
            pnwn, pnbn = blocks[i + 1][0], blocks[i + 1][1]
            x, xn128 = _upd_mid(agg2a, agg2b, x, xn128, wmsg, wupd, wself,
                                rs.reshape(1, 1), pnwn.reshape(1, DNODE),
                                pnbn.reshape(1, DNODE))
        else:
            y = _upd_last(agg2a, agg2b, x, xn128, wmsg, wupd, wself,
                          rs.reshape(1, 1), maskb)
    return y.reshape(B, N, DNODE) * 1.0


# speedup vs baseline: 1.0440x; 1.0440x over previous
import functools

import jax
import jax.numpy as jnp
import numpy as np
from jax import lax
from jax.experimental import pallas as pl
from jax.experimental.pallas import tpu as pltpu
from jax.experimental.pallas import tpu_sc as plsc

_NC = 2
_NS = 16
_NW = _NC * _NS

B = 32
N = 32
NTOT = B * N
CUTOFF = 6.0
NBLK = 3
RBF = 16
HID = 128
DNODE = 64
DMSG = 16

_DELTA = CUTOFF / (RBF - 1)
_GAMMA = 1.0 / (_DELTA * _DELTA + 1e-12)

_R_NP = np.zeros((DNODE, DNODE * DMSG), np.float32)
for _i in range(DNODE):
    _R_NP[_i, _i * DMSG:(_i + 1) * DMSG] = 1.0
_S_NP = np.zeros((DNODE * DMSG, DMSG), np.float32)
for _j in range(DNODE * DMSG):
    _S_NP[_j, _j % DMSG] = 1.0

_TE = 2048

_COSPOLY = (1.00000000e+00, -4.93480220e+00, 4.05871213e+00,
            -1.33526277e+00, 2.35330630e-01, -2.58068912e-02,
            1.92957371e-03, -1.04637021e-04, 4.30179273e-06,
            -1.37843422e-07, 3.20231643e-09)


def _silu(x):
    return x / (1.0 + jnp.exp(-x))


def _full(shape):
    return pl.BlockSpec(shape, lambda i: (0, 0))


def _geom_body(E, es_ref, ed_ref, poshi_ref, poslo_ref, nid_ref, out_ref):
    i = pl.program_id(0)
    bf16 = jnp.bfloat16
    f32 = jnp.float32
    nid = nid_ref[...]
    dsel = ((nid == ed_ref[...]).astype(bf16)
            - (nid == es_ref[...]).astype(bf16))
    d = (jnp.dot(dsel, poshi_ref[...], preferred_element_type=f32)
         + jnp.dot(dsel, poslo_ref[...], preferred_element_type=f32))
    r = jnp.sqrt(jnp.sum(d * d, axis=1, keepdims=True) + 1e-16)
    centers = lax.broadcasted_iota(
        jnp.int32, (_TE, RBF), 1).astype(jnp.float32) * _DELTA
    rbf = jnp.exp(-_GAMMA * (r - centers) ** 2)
    row = lax.broadcasted_iota(jnp.int32, (_TE, 1), 0) + i * _TE
    valid = (row < E).astype(jnp.float32)
    u = r * r * (1.0 / (CUTOFF * CUTOFF))
    c = jnp.float32(_COSPOLY[-1])
    for a in _COSPOLY[-2::-1]:
        c = c * u + jnp.float32(a)
    cut = 0.5 * (c + 1.0)
    cut = cut * (r <= CUTOFF).astype(jnp.float32) * valid
    out_ref[...] = jnp.concatenate(
        [rbf, jnp.broadcast_to(cut, (_TE, RBF))], axis=1).astype(jnp.bfloat16)


def _geom(esc, edc, poshi, poslo, nid, E, Epad):
    return pl.pallas_call(
        functools.partial(_geom_body, E),
        grid=(Epad // _TE,),
        in_specs=[pl.BlockSpec((_TE, 1), lambda i: (i, 0)),
                  pl.BlockSpec((_TE, 1), lambda i: (i, 0)),
                  pl.BlockSpec((NTOT, 16), lambda i: (0, 0)),
                  pl.BlockSpec((NTOT, 16), lambda i: (0, 0)),
                  pl.BlockSpec((1, NTOT), lambda i: (0, 0))],
        out_specs=pl.BlockSpec((_TE, 2 * RBF), lambda i: (i, 0)),
        out_shape=jax.ShapeDtypeStruct((Epad, 2 * RBF), jnp.bfloat16),
    )(esc, edc, poshi, poslo, nid)


def _edge_body(rc_ref, es_ref, xn_ref, nid_ref, rw1, rw2, rw3,
               gw1, gw2, rm, sm, out_ref):
    f32 = jnp.float32
    bf16 = jnp.bfloat16
    rc = rc_ref[...]
    rbf = rc[:, :RBF]
    cut = rc[:, RBF:RBF + 1].astype(f32)
    h = _silu(jnp.dot(rbf, rw1[...], preferred_element_type=f32))
    h = _silu(jnp.dot(h.astype(bf16), rw2[...], preferred_element_type=f32))
    tw = jnp.dot(h.astype(bf16), rw3[...], preferred_element_type=f32)
    gs = (nid_ref[...] == es_ref[...]).astype(bf16)
    xse = jnp.dot(gs, xn_ref[...][:, :DNODE].astype(bf16),
                  preferred_element_type=f32)
    xr = jnp.dot(xse.astype(bf16), rm[...], preferred_element_type=f32)
    m = jnp.dot((tw * xr).astype(bf16), sm[...],
                preferred_element_type=f32) * 0.125
    g = _silu(jnp.dot(rbf, gw1[...], preferred_element_type=f32))
    gz = jnp.dot(g.astype(bf16), gw2[...], preferred_element_type=f32)
    sg = 1.0 / (1.0 + jnp.exp(-gz[:, :1]))
    ew = cut * sg
    out_ref[...] = jnp.concatenate(
        [m * ew, jnp.broadcast_to(ew, (_TE, 16)),
         jnp.zeros((_TE, 96), f32)], axis=1)


def _edge(rc, esc, xn128, nid, w, e0, Ec):
    (rw1, rw2, rw3, gw1, gw2, rm, sm) = w
    b0 = e0 // _TE
    return pl.pallas_call(
        _edge_body,
        grid=(Ec // _TE,),
        in_specs=[pl.BlockSpec((_TE, 2 * RBF), lambda i: (b0 + i, 0)),
                  pl.BlockSpec((_TE, 1), lambda i: (b0 + i, 0)),
                  _full((NTOT, 128)),
                  pl.BlockSpec((1, NTOT), lambda i: (0, 0)),
                  _full((RBF, HID)),
                  _full((HID, HID)),
                  _full((HID, DNODE * DMSG)),
                  _full((RBF, HID)),
                  _full((HID, HID)),
                  _full((DNODE, DNODE * DMSG)),
                  _full((DNODE * DMSG, DMSG))],
        out_specs=pl.BlockSpec((_TE, 128), lambda i: (i, 0)),
        out_shape=jax.ShapeDtypeStruct((Ec, 128), jnp.float32),
    )(rc, esc, xn128, nid, rw1, rw2, rw3, gw1, gw2, rm, sm)


def _ln(x, w, b):
    mu = jnp.mean(x, axis=1, keepdims=True)
    var = jnp.mean((x - mu) ** 2, axis=1, keepdims=True)
    return (x - mu) / jnp.sqrt(var + 1e-8) * w + b


def _ln0_body(x_ref, w_ref, b_ref, out_ref):
    xn = _ln(x_ref[...], w_ref[...], b_ref[...])
    out_ref[...] = jnp.concatenate(
        [xn, jnp.zeros((NTOT, 128 - DNODE), jnp.float32)], axis=1)


def _ln0(x, w, b):
    return pl.pallas_call(
        _ln0_body,
        out_shape=jax.ShapeDtypeStruct((NTOT, 128), jnp.float32),
    )(x, w, b)


def _upd_core(agg2a_ref, agg2b_ref, x_ref, xn_ref, wmsg, wupd, wself,
              rs_ref):
    f32 = jnp.float32
    s = (agg2a_ref[0] + agg2a_ref[1]) + (agg2b_ref[0] + agg2b_ref[1])
    agg = s[:, :DMSG]
    nrm = s[:, DMSG:DMSG + 1]
    agg = agg / jnp.maximum(nrm, 1e-8)
    agg = _silu(jnp.dot(agg, wmsg[...], preferred_element_type=f32) * 0.25)
    xn = xn_ref[...][:, :DNODE]
    out = (jnp.dot(xn, wself[...], preferred_element_type=f32) * 0.125
           + jnp.dot(agg, wupd[...], preferred_element_type=f32) * 0.25)
    return x_ref[...] + rs_ref[0, 0] * out


def _upd_mid_body(agg2a_ref, agg2b_ref, x_ref, xn_ref, wmsg, wupd, wself,
                  rs_ref, pnw_ref, pnb_ref, xo_ref, xno_ref):
    xnew = _upd_core(agg2a_ref, agg2b_ref, x_ref, xn_ref, wmsg, wupd,
                     wself, rs_ref)
    xo_ref[...] = xnew
    xn2 = _ln(xnew, pnw_ref[...], pnb_ref[...])
    xno_ref[...] = jnp.concatenate(
        [xn2, jnp.zeros((NTOT, 128 - DNODE), jnp.float32)], axis=1)


def _upd_last_body(agg2a_ref, agg2b_ref, x_ref, xn_ref, wmsg, wupd, wself,
                   rs_ref, mask_ref, y_ref):
    xnew = _upd_core(agg2a_ref, agg2b_ref, x_ref, xn_ref, wmsg, wupd,
                     wself, rs_ref)
    y_ref[...] = xnew * mask_ref[...]


def _upd_mid(agg2a, agg2b, x, xn128, wmsg, wupd, wself, rs, pnw, pnb):
    return pl.pallas_call(
        _upd_mid_body,
        out_shape=[jax.ShapeDtypeStruct((NTOT, DNODE), jnp.float32),
                   jax.ShapeDtypeStruct((NTOT, 128), jnp.float32)],
    )(agg2a, agg2b, x, xn128, wmsg, wupd, wself, rs, pnw, pnb)


def _upd_last(agg2a, agg2b, x, xn128, wmsg, wupd, wself, rs, maskb):
    return pl.pallas_call(
        _upd_last_body,
        out_shape=jax.ShapeDtypeStruct((NTOT, DNODE), jnp.float32),
    )(agg2a, agg2b, x, xn128, wmsg, wupd, wself, rs, maskb)


def _sc_mesh():
    return plsc.VectorSubcoreMesh(core_axis_name="c", subcore_axis_name="s",
                                  num_cores=_NC, num_subcores=_NS)


def _wid():
    return lax.axis_index("s") * _NC + lax.axis_index("c")


def _sc_prep(embed128, zf):
    f32 = jnp.float32
    nz = NTOT // _NW

    @functools.partial(
        pl.kernel,
        out_type=jax.ShapeDtypeStruct((NTOT, 128), f32),
        mesh=_sc_mesh(),
        scratch_types=[pltpu.VMEM((nz,), jnp.int32),
                       pltpu.VMEM((nz, 128), f32),
                       pltpu.SemaphoreType.DMA],
    )
    def prep(embed_h, zf_h, x0_h, zbuf, zrows, sem):
        wid = _wid()
        pltpu.sync_copy(zf_h.at[pl.ds(wid * nz, nz)], zbuf)
        pltpu.async_copy(embed_h.at[zbuf], zrows, sem).wait()
        pltpu.sync_copy(zrows, x0_h.at[pl.ds(wid * nz, nz)])

    return prep(embed128, zf)


def _sc_scatter(medge, edp2, zeros128, e0, Ec):
    K = Ec // (_NW * 128)
    r0 = e0 // 128
    f32 = jnp.float32
    nrows = NTOT // _NS

    @functools.partial(
        pl.kernel,
        out_type=jax.ShapeDtypeStruct((_NC, NTOT, 128), f32),
        mesh=_sc_mesh(),
        scratch_types=[pltpu.VMEM_SHARED((NTOT, 128), f32),
                       pltpu.VMEM((K, 128), jnp.int32),
                       pltpu.VMEM((128, 128), f32),
                       pltpu.VMEM((128, 128), f32),
                       pltpu.SemaphoreType.DMA,
                       pltpu.SemaphoreType.DMA],
    )
    def scat(m_h, ed_h, z_h, out_h, shared, idx_v, mb0, mb1, lsem, asem):
        cid = lax.axis_index("c")
        sid = lax.axis_index("s")
        wid = sid * _NC + cid
        bufs = (mb0, mb1)
        pltpu.sync_copy(z_h.at[pl.ds(sid * nrows, nrows)],
                        shared.at[pl.ds(sid * nrows, nrows)])
        plsc.subcore_barrier()
        pltpu.sync_copy(ed_h.at[pl.ds(r0 + wid * K, K)], idx_v)
        ld = pltpu.async_copy(m_h.at[pl.ds(wid * K * 128, 128)], bufs[0], lsem)
        ad = None
        for j in range(K):
            ld.wait()
            if ad is not None:
                ad.wait()
            if j + 1 < K:
                ld = pltpu.async_copy(
                    m_h.at[pl.ds((wid * K + j + 1) * 128, 128)],
                    bufs[(j + 1) % 2], lsem)
            ad = pltpu.async_copy(
                bufs[j % 2], shared.at[idx_v.at[j]], asem, add=True)
        ad.wait()
        plsc.subcore_barrier()
        pltpu.sync_copy(shared.at[pl.ds(sid * nrows, nrows)],
                        out_h.at[cid, pl.ds(sid * nrows, nrows)])

    return scat(medge, edp2, zeros128)


def kernel(z, pos, mask, edge_src, edge_dst, embed,
           pnw0, pnb0, rw10, rb10, rw20, rb20, rw30, rb30,
           gw10, gb10, gw20, gb20, wmsg0, wupd0, wself0, rs0,
           pnw1, pnb1, rw11, rb11, rw21, rb21, rw31, rb31,
           gw11, gb11, gw21, gb21, wmsg1, wupd1, wself1, rs1,
           pnw2, pnb2, rw12, rb12, rw22, rb22, rw32, rb32,
           gw12, gb12, gw22, gb22, wmsg2, wupd2, wself2, rs2):
    f32 = jnp.float32
    E = edge_src.shape[0]
    Epad = max(((E + 8191) // 8192) * 8192, 8192)
    Ec = Epad // 2
    esp = jnp.pad(edge_src.astype(jnp.int32), (0, Epad - E))
    edp = jnp.pad(edge_dst.astype(jnp.int32), (0, Epad - E))
    edp2 = edp.reshape(-1, 128)
    esc = esp.reshape(-1, 1)
    edc = edp.reshape(-1, 1)
    posf = pos.reshape(NTOT, 3).astype(f32)
    posp = jnp.pad(posf, ((0, 0), (0, 13)))
    poshi = posp.astype(jnp.bfloat16)
    poslo = (posp - poshi.astype(f32)).astype(jnp.bfloat16)
    embed128 = jnp.pad(embed.astype(f32), ((0, 0), (0, 128 - DNODE)))
    zf = z.reshape(-1).astype(jnp.int32)
    zeros128 = jnp.zeros((NTOT, 128), f32)
    maskb = jnp.broadcast_to(mask.reshape(NTOT, 1), (NTOT, DNODE)).astype(f32)
    rm = jnp.asarray(_R_NP, dtype=jnp.bfloat16)
    sm = jnp.asarray(_S_NP, dtype=jnp.bfloat16)
    nid = jnp.arange(NTOT, dtype=jnp.int32).reshape(1, NTOT)

    blocks = [
        (pnw0, pnb0, rw10, rb10, rw20, rb20, rw30, rb30, gw10, gb10, gw20,
         gb20, wmsg0, wupd0, wself0, rs0),
        (pnw1, pnb1, rw11, rb11, rw21, rb21, rw31, rb31, gw11, gb11, gw21,
         gb21, wmsg1, wupd1, wself1, rs1),
        (pnw2, pnb2, rw12, rb12, rw22, rb22, rw32, rb32, gw12, gb12, gw22,
         gb22, wmsg2, wupd2, wself2, rs2),
    ]

    x0128 = _sc_prep(embed128, zf)
    x = x0128[:, :DNODE]

    rc = _geom(esc, edc, poshi, poslo, nid, E, Epad)

    xn128 = _ln0(x, pnw0.reshape(1, DNODE), pnb0.reshape(1, DNODE))
    for i in range(NBLK):
        (pnw, pnb, rw1, rb1, rw2, rb2, rw3, rb3, gw1, gb1, gw2, gb2,
         wmsg, wupd, wself, rs) = blocks[i]
        bf16 = jnp.bfloat16
        gw2p = jnp.pad(gw2, ((0, 0), (0, HID - 1))).astype(bf16)
        w = (rw1.astype(bf16), rw2.astype(bf16), rw3.astype(bf16),
             gw1.astype(bf16), gw2p, rm, sm)
        medge0 = _edge(rc, esc, xn128, nid, w, 0, Ec)
        agg2a = _sc_scatter(medge0, edp2, zeros128, 0, Ec)
        medge1 = _edge(rc, esc, xn128, nid, w, Ec, Ec)
        agg2b = _sc_scatter(medge1, edp2, zeros128, Ec, Ec)
        if i < NBLK - 1:
            pnwn, pnbn = blocks[i + 1][0], blocks[i + 1][1]
            x, xn128 = _upd_mid(agg2a, agg2b, x, xn128, wmsg, wupd, wself,
                                rs.reshape(1, 1), pnwn.reshape(1, DNODE),
                                pnbn.reshape(1, DNODE))
        else:
            y = _upd_last(agg2a, agg2b, x, xn128, wmsg, wupd, wself,
                          rs.reshape(1, 1), maskb)
    return y.reshape(B, N, DNODE) * 1.0


# speedup vs baseline: 1.0440x; 1.0000x over previous
import functools

import jax
import jax.numpy as jnp
import numpy as np
from jax import lax
from jax.experimental import pallas as pl
from jax.experimental.pallas import tpu as pltpu
from jax.experimental.pallas import tpu_sc as plsc

_NC = 2
_NS = 16
_NW = _NC * _NS

B = 32
N = 32
NTOT = B * N
CUTOFF = 6.0
NBLK = 3
RBF = 16
HID = 128
DNODE = 64
DMSG = 16

_DELTA = CUTOFF / (RBF - 1)
_GAMMA = 1.0 / (_DELTA * _DELTA + 1e-12)

_R_NP = np.zeros((DNODE, DNODE * DMSG), np.float32)
for _i in range(DNODE):
    _R_NP[_i, _i * DMSG:(_i + 1) * DMSG] = 1.0
_S_NP = np.zeros((DNODE * DMSG, DMSG), np.float32)
for _j in range(DNODE * DMSG):
    _S_NP[_j, _j % DMSG] = 1.0

_TE = 4096

_COSPOLY = (1.00000000e+00, -4.93480220e+00, 4.05871213e+00,
            -1.33526277e+00, 2.35330630e-01, -2.58068912e-02,
            1.92957371e-03, -1.04637021e-04, 4.30179273e-06,
            -1.37843422e-07, 3.20231643e-09)


def _silu(x):
    return x / (1.0 + jnp.exp(-x))


def _full(shape):
    return pl.BlockSpec(shape, lambda i: (0, 0))


def _geom_body(E, es_ref, ed_ref, poshi_ref, poslo_ref, nid_ref, out_ref):
    i = pl.program_id(0)
    bf16 = jnp.bfloat16
    f32 = jnp.float32
    nid = nid_ref[...]
    dsel = ((nid == ed_ref[...]).astype(bf16)
            - (nid == es_ref[...]).astype(bf16))
    d = (jnp.dot(dsel, poshi_ref[...], preferred_element_type=f32)
         + jnp.dot(dsel, poslo_ref[...], preferred_element_type=f32))
    r = jnp.sqrt(jnp.sum(d * d, axis=1, keepdims=True) + 1e-16)
    centers = lax.broadcasted_iota(
        jnp.int32, (_TE, RBF), 1).astype(jnp.float32) * _DELTA
    rbf = jnp.exp(-_GAMMA * (r - centers) ** 2)
    row = lax.broadcasted_iota(jnp.int32, (_TE, 1), 0) + i * _TE
    valid = (row < E).astype(jnp.float32)
    u = r * r * (1.0 / (CUTOFF * CUTOFF))
    c = jnp.float32(_COSPOLY[-1])
    for a in _COSPOLY[-2::-1]:
        c = c * u + jnp.float32(a)
    cut = 0.5 * (c + 1.0)
    cut = cut * (r <= CUTOFF).astype(jnp.float32) * valid
    out_ref[...] = jnp.concatenate(
        [rbf, jnp.broadcast_to(cut, (_TE, RBF))], axis=1).astype(jnp.bfloat16)


def _geom(esc, edc, poshi, poslo, nid, E, Epad):
    return pl.pallas_call(
        functools.partial(_geom_body, E),
        grid=(Epad // _TE,),
        in_specs=[pl.BlockSpec((_TE, 1), lambda i: (i, 0)),
                  pl.BlockSpec((_TE, 1), lambda i: (i, 0)),
                  pl.BlockSpec((NTOT, 16), lambda i: (0, 0)),
                  pl.BlockSpec((NTOT, 16), lambda i: (0, 0)),
                  pl.BlockSpec((1, NTOT), lambda i: (0, 0))],
        out_specs=pl.BlockSpec((_TE, 2 * RBF), lambda i: (i, 0)),
        out_shape=jax.ShapeDtypeStruct((Epad, 2 * RBF), jnp.bfloat16),
    )(esc, edc, poshi, poslo, nid)


def _edge_body(rc_ref, es_ref, xn_ref, nid_ref, rw1, rw2, rw3,
               gw1, gw2, rm, sm, out_ref):
    f32 = jnp.float32
    bf16 = jnp.bfloat16
    rc = rc_ref[...]
    rbf = rc[:, :RBF]
    cut = rc[:, RBF:RBF + 1].astype(f32)
    h = _silu(jnp.dot(rbf, rw1[...], preferred_element_type=f32))
    h = _silu(jnp.dot(h.astype(bf16), rw2[...], preferred_element_type=f32))
    tw = jnp.dot(h.astype(bf16), rw3[...], preferred_element_type=f32)
    gs = (nid_ref[...] == es_ref[...]).astype(bf16)
    xse = jnp.dot(gs, xn_ref[...][:, :DNODE].astype(bf16),
                  preferred_element_type=f32)
    xr = jnp.dot(xse.astype(bf16), rm[...], preferred_element_type=f32)
    m = jnp.dot((tw * xr).astype(bf16), sm[...],
                preferred_element_type=f32) * 0.125
    g = _silu(jnp.dot(rbf, gw1[...], preferred_element_type=f32))
    gz = jnp.dot(g.astype(bf16), gw2[...], preferred_element_type=f32)
    sg = 1.0 / (1.0 + jnp.exp(-gz[:, :1]))
    ew = cut * sg
    out_ref[...] = jnp.concatenate(
        [m * ew, jnp.broadcast_to(ew, (_TE, 16)),
         jnp.zeros((_TE, 96), f32)], axis=1)


def _edge(rc, esc, xn128, nid, w, e0, Ec):
    (rw1, rw2, rw3, gw1, gw2, rm, sm) = w
    b0 = e0 // _TE
    return pl.pallas_call(
        _edge_body,
        grid=(Ec // _TE,),
        in_specs=[pl.BlockSpec((_TE, 2 * RBF), lambda i: (b0 + i, 0)),
                  pl.BlockSpec((_TE, 1), lambda i: (b0 + i, 0)),
                  _full((NTOT, 128)),
                  pl.BlockSpec((1, NTOT), lambda i: (0, 0)),
                  _full((RBF, HID)),
                  _full((HID, HID)),
                  _full((HID, DNODE * DMSG)),
                  _full((RBF, HID)),
                  _full((HID, HID)),
                  _full((DNODE, DNODE * DMSG)),
                  _full((DNODE * DMSG, DMSG))],
        out_specs=pl.BlockSpec((_TE, 128), lambda i: (i, 0)),
        out_shape=jax.ShapeDtypeStruct((Ec, 128), jnp.float32),
    )(rc, esc, xn128, nid, rw1, rw2, rw3, gw1, gw2, rm, sm)


def _ln(x, w, b):
    mu = jnp.mean(x, axis=1, keepdims=True)
    var = jnp.mean((x - mu) ** 2, axis=1, keepdims=True)
    return (x - mu) / jnp.sqrt(var + 1e-8) * w + b


def _ln0_body(x_ref, w_ref, b_ref, out_ref):
    xn = _ln(x_ref[...], w_ref[...], b_ref[...])
    out_ref[...] = jnp.concatenate(
        [xn, jnp.zeros((NTOT, 128 - DNODE), jnp.float32)], axis=1)


def _ln0(x, w, b):
    return pl.pallas_call(
        _ln0_body,
        out_shape=jax.ShapeDtypeStruct((NTOT, 128), jnp.float32),
    )(x, w, b)


def _upd_core(agg2a_ref, agg2b_ref, x_ref, xn_ref, wmsg, wupd, wself,
              rs_ref):
    f32 = jnp.float32
    s = (agg2a_ref[0] + agg2a_ref[1]) + (agg2b_ref[0] + agg2b_ref[1])
    agg = s[:, :DMSG]
    nrm = s[:, DMSG:DMSG + 1]
    agg = agg / jnp.maximum(nrm, 1e-8)
    agg = _silu(jnp.dot(agg, wmsg[...], preferred_element_type=f32) * 0.25)
    xn = xn_ref[...][:, :DNODE]
    out = (jnp.dot(xn, wself[...], preferred_element_type=f32) * 0.125
           + jnp.dot(agg, wupd[...], preferred_element_type=f32) * 0.25)
    return x_ref[...] + rs_ref[0, 0] * out


def _upd_mid_body(agg2a_ref, agg2b_ref, x_ref, xn_ref, wmsg, wupd, wself,
                  rs_ref, pnw_ref, pnb_ref, xo_ref, xno_ref):
    xnew = _upd_core(agg2a_ref, agg2b_ref, x_ref, xn_ref, wmsg, wupd,
                     wself, rs_ref)
    xo_ref[...] = xnew
    xn2 = _ln(xnew, pnw_ref[...], pnb_ref[...])
    xno_ref[...] = jnp.concatenate(
        [xn2, jnp.zeros((NTOT, 128 - DNODE), jnp.float32)], axis=1)


def _upd_last_body(agg2a_ref, agg2b_ref, x_ref, xn_ref, wmsg, wupd, wself,
                   rs_ref, mask_ref, y_ref):
    xnew = _upd_core(agg2a_ref, agg2b_ref, x_ref, xn_ref, wmsg, wupd,
                     wself, rs_ref)
    y_ref[...] = xnew * mask_ref[...]


def _upd_mid(agg2a, agg2b, x, xn128, wmsg, wupd, wself, rs, pnw, pnb):
    return pl.pallas_call(
        _upd_mid_body,
        out_shape=[jax.ShapeDtypeStruct((NTOT, DNODE), jnp.float32),
                   jax.ShapeDtypeStruct((NTOT, 128), jnp.float32)],
    )(agg2a, agg2b, x, xn128, wmsg, wupd, wself, rs, pnw, pnb)


def _upd_last(agg2a, agg2b, x, xn128, wmsg, wupd, wself, rs, maskb):
    return pl.pallas_call(
        _upd_last_body,
        out_shape=jax.ShapeDtypeStruct((NTOT, DNODE), jnp.float32),
    )(agg2a, agg2b, x, xn128, wmsg, wupd, wself, rs, maskb)


def _sc_mesh():
    return plsc.VectorSubcoreMesh(core_axis_name="c", subcore_axis_name="s",
                                  num_cores=_NC, num_subcores=_NS)


def _wid():
    return lax.axis_index("s") * _NC + lax.axis_index("c")


def _sc_prep(embed128, zf):
    f32 = jnp.float32
    nz = NTOT // _NW

    @functools.partial(
        pl.kernel,
        out_type=jax.ShapeDtypeStruct((NTOT, 128), f32),
        mesh=_sc_mesh(),
        scratch_types=[pltpu.VMEM((nz,), jnp.int32),
                       pltpu.VMEM((nz, 128), f32),
                       pltpu.SemaphoreType.DMA],
    )
    def prep(embed_h, zf_h, x0_h, zbuf, zrows, sem):
        wid = _wid()
        pltpu.sync_copy(zf_h.at[pl.ds(wid * nz, nz)], zbuf)
        pltpu.async_copy(embed_h.at[zbuf], zrows, sem).wait()
        pltpu.sync_copy(zrows, x0_h.at[pl.ds(wid * nz, nz)])

    return prep(embed128, zf)


def _sc_scatter(medge, edp2, zeros128, e0, Ec):
    K = Ec // (_NW * 128)
    r0 = e0 // 128
    f32 = jnp.float32
    nrows = NTOT // _NS

    @functools.partial(
        pl.kernel,
        out_type=jax.ShapeDtypeStruct((_NC, NTOT, 128), f32),
        mesh=_sc_mesh(),
        scratch_types=[pltpu.VMEM_SHARED((NTOT, 128), f32),
                       pltpu.VMEM((K, 128), jnp.int32),
                       pltpu.VMEM((128, 128), f32),
                       pltpu.VMEM((128, 128), f32),
                       pltpu.SemaphoreType.DMA,
                       pltpu.SemaphoreType.DMA],
    )
    def scat(m_h, ed_h, z_h, out_h, shared, idx_v, mb0, mb1, lsem, asem):
        cid = lax.axis_index("c")
        sid = lax.axis_index("s")
        wid = sid * _NC + cid
        bufs = (mb0, mb1)
        pltpu.sync_copy(z_h.at[pl.ds(sid * nrows, nrows)],
                        shared.at[pl.ds(sid * nrows, nrows)])
        plsc.subcore_barrier()
        pltpu.sync_copy(ed_h.at[pl.ds(r0 + wid * K, K)], idx_v)
        ld = pltpu.async_copy(m_h.at[pl.ds(wid * K * 128, 128)], bufs[0], lsem)
        ad = None
        for j in range(K):
            ld.wait()
            if ad is not None:
                ad.wait()
            if j + 1 < K:
                ld = pltpu.async_copy(
                    m_h.at[pl.ds((wid * K + j + 1) * 128, 128)],
                    bufs[(j + 1) % 2], lsem)
            ad = pltpu.async_copy(
                bufs[j % 2], shared.at[idx_v.at[j]], asem, add=True)
        ad.wait()
        plsc.subcore_barrier()
        pltpu.sync_copy(shared.at[pl.ds(sid * nrows, nrows)],
                        out_h.at[cid, pl.ds(sid * nrows, nrows)])

    return scat(medge, edp2, zeros128)


def kernel(z, pos, mask, edge_src, edge_dst, embed,
           pnw0, pnb0, rw10, rb10, rw20, rb20, rw30, rb30,
           gw10, gb10, gw20, gb20, wmsg0, wupd0, wself0, rs0,
           pnw1, pnb1, rw11, rb11, rw21, rb21, rw31, rb31,
           gw11, gb11, gw21, gb21, wmsg1, wupd1, wself1, rs1,
           pnw2, pnb2, rw12, rb12, rw22, rb22, rw32, rb32,
           gw12, gb12, gw22, gb22, wmsg2, wupd2, wself2, rs2):
    f32 = jnp.float32
    E = edge_src.shape[0]
    Epad = max(((E + 8191) // 8192) * 8192, 8192)
    Ec = Epad // 2
    esp = jnp.pad(edge_src.astype(jnp.int32), (0, Epad - E))
    edp = jnp.pad(edge_dst.astype(jnp.int32), (0, Epad - E))
    edp2 = edp.reshape(-1, 128)
    esc = esp.reshape(-1, 1)
    edc = edp.reshape(-1, 1)
    posf = pos.reshape(NTOT, 3).astype(f32)
    posp = jnp.pad(posf, ((0, 0), (0, 13)))
    poshi = posp.astype(jnp.bfloat16)
    poslo = (posp - poshi.astype(f32)).astype(jnp.bfloat16)
    embed128 = jnp.pad(embed.astype(f32), ((0, 0), (0, 128 - DNODE)))
    zf = z.reshape(-1).astype(jnp.int32)
    zeros128 = jnp.zeros((NTOT, 128), f32)
    maskb = jnp.broadcast_to(mask.reshape(NTOT, 1), (NTOT, DNODE)).astype(f32)
    rm = jnp.asarray(_R_NP, dtype=jnp.bfloat16)
    sm = jnp.asarray(_S_NP, dtype=jnp.bfloat16)
    nid = jnp.arange(NTOT, dtype=jnp.int32).reshape(1, NTOT)

    blocks = [
        (pnw0, pnb0, rw10, rb10, rw20, rb20, rw30, rb30, gw10, gb10, gw20,
         gb20, wmsg0, wupd0, wself0, rs0),
        (pnw1, pnb1, rw11, rb11, rw21, rb21, rw31, rb31, gw11, gb11, gw21,
         gb21, wmsg1, wupd1, wself1, rs1),
        (pnw2, pnb2, rw12, rb12, rw22, rb22, rw32, rb32, gw12, gb12, gw22,
         gb22, wmsg2, wupd2, wself2, rs2),
    ]

    x0128 = _sc_prep(embed128, zf)
    x = x0128[:, :DNODE]

    rc = _geom(esc, edc, poshi, poslo, nid, E, Epad)

    xn128 = _ln0(x, pnw0.reshape(1, DNODE), pnb0.reshape(1, DNODE))
    for i in range(NBLK):
        (pnw, pnb, rw1, rb1, rw2, rb2, rw3, rb3, gw1, gb1, gw2, gb2,
         wmsg, wupd, wself, rs) = blocks[i]
        bf16 = jnp.bfloat16
        gw2p = jnp.pad(gw2, ((0, 0), (0, HID - 1))).astype(bf16)
        w = (rw1.astype(bf16), rw2.astype(bf16), rw3.astype(bf16),
             gw1.astype(bf16), gw2p, rm, sm)
        medge0 = _edge(rc, esc, xn128, nid, w, 0, Ec)
        agg2a = _sc_scatter(medge0, edp2, zeros128, 0, Ec)
        medge1 = _edge(rc, esc, xn128, nid, w, Ec, Ec)
        agg2b = _sc_scatter(medge1, edp2, zeros128, Ec, Ec)
        if i < NBLK - 1:
            pnwn, pnbn = blocks[i + 1][0], blocks[i + 1][1]
            x, xn128 = _upd_mid(agg2a, agg2b, x, xn128, wmsg, wupd, wself,
                                rs.reshape(1, 1), pnwn.reshape(1, DNODE),
                                pnbn.reshape(1, DNODE))
        else:
            y = _upd_last(agg2a, agg2b, x, xn128, wmsg, wupd, wself,
                          rs.reshape(1, 1), maskb)
    return y.reshape(B, N, DNODE) * 1.0
